# baseline (speedup 1.0000x reference)
import functools

import jax
import jax.numpy as jnp
from jax import lax
from jax.experimental import pallas as pl
from jax.experimental.pallas import tpu as pltpu
from jax.experimental.pallas import tpu_sc as plsc

NC = 2
NS = 16
NW = NC * NS


def _sc_gather_call(idx_flat, table_flat, B, CAT_F, CAT_DIM, CARD1):
    TOT = B * CAT_F
    per_w = TOT // NW
    CHUNKS = 2
    RC = per_w // CHUNKS

    mesh = plsc.VectorSubcoreMesh(core_axis_name="c", subcore_axis_name="s")

    @functools.partial(
        pl.kernel,
        mesh=mesh,
        out_type=jax.ShapeDtypeStruct((TOT, CAT_DIM), jnp.float32),
        compiler_params=pltpu.CompilerParams(use_tc_tiling_on_sc=False),
        scratch_types=[
            pltpu.VMEM((RC,), jnp.int32),
            pltpu.VMEM((RC,), jnp.int32),
            pltpu.VMEM((RC, CAT_DIM), jnp.float32),
            pltpu.SemaphoreType.DMA,
        ],
    )
    def gather_kernel(idx_hbm, table_hbm, out_hbm, idx_v, gidx_v, rows_v, sem):
        wid = lax.axis_index("s") * NC + lax.axis_index("c")
        lane = jnp.arange(16, dtype=jnp.int32)
        for c in range(CHUNKS):
            base = wid * per_w + c * RC
            pltpu.sync_copy(idx_hbm.at[pl.ds(base, RC)], idx_v)

            def body(j, _):
                o = pl.multiple_of(j * 16, 16)
                pos = lane + j * 16
                f = lax.rem(pos, CAT_F)
                gidx_v[pl.ds(o, 16)] = idx_v[pl.ds(o, 16)] + f * CARD1
                return 0

            lax.fori_loop(0, RC // 16, body, 0)
            pltpu.async_copy(table_hbm.at[gidx_v], rows_v, sem).wait()
            pltpu.sync_copy(rows_v, out_hbm.at[pl.ds(base, RC)])

    return gather_kernel(idx_flat, table_flat)


def _tc_numerical_call(vals, W_flat, b_flat, NUM_F, NUM_DIM):
    B = vals.shape[0]
    OUT = NUM_F * NUM_DIM
    BLK = 2048

    def num_kernel(v_ref, w_ref, b_ref, o_ref):
        v = v_ref[...]
        m = jnp.logical_not(jnp.isnan(v))
        mf = m.astype(jnp.float32)
        v0 = jnp.where(m, v, 0.0)
        ci = lax.broadcasted_iota(jnp.int32, (NUM_F, OUT), 1) // NUM_DIM
        ri = lax.broadcasted_iota(jnp.int32, (NUM_F, OUT), 0)
        S = (ci == ri).astype(jnp.float32)
        v_rep = jax.lax.dot(v0, S, precision=lax.Precision.HIGHEST)
        m_rep = jax.lax.dot(mf, S, precision=lax.Precision.HIGHEST)
        o_ref[...] = jnp.maximum(v_rep * w_ref[...] + b_ref[...], 0.0) * m_rep

    return pl.pallas_call(
        num_kernel,
        grid=(B // BLK,),
        in_specs=[
            pl.BlockSpec((BLK, NUM_F), lambda i: (i, 0)),
            pl.BlockSpec((1, OUT), lambda i: (0, 0)),
            pl.BlockSpec((1, OUT), lambda i: (0, 0)),
        ],
        out_specs=pl.BlockSpec((BLK, OUT), lambda i: (i, 0)),
        out_shape=jax.ShapeDtypeStruct((B, OUT), jnp.float32),
    )(vals, W_flat, b_flat)


def kernel(numerical_values, categorical_indices, W_num, b_num, emb_tables):
    B, NUM_F = numerical_values.shape
    CAT_F = categorical_indices.shape[1]
    CARD1 = emb_tables.shape[1]
    CAT_DIM = emb_tables.shape[2]
    NUM_DIM = W_num.shape[1]

    table_flat = emb_tables.reshape(CAT_F * CARD1, CAT_DIM)
    idx_flat = categorical_indices.astype(jnp.int32).reshape(B * CAT_F)

    cat = _sc_gather_call(idx_flat, table_flat, B, CAT_F, CAT_DIM, CARD1)
    cat = cat.reshape(B, CAT_F * CAT_DIM)

    W_flat = W_num.reshape(1, NUM_F * NUM_DIM)
    b_flat = b_num.reshape(1, NUM_F * NUM_DIM)
    num = _tc_numerical_call(numerical_values, W_flat, b_flat, NUM_F, NUM_DIM)

    return jnp.concatenate([num, cat], axis=1)


# speedup vs baseline: 2.6003x; 2.6003x over previous
import functools

import jax
import jax.numpy as jnp
from jax import lax
from jax.experimental import pallas as pl
from jax.experimental.pallas import tpu as pltpu
from jax.experimental.pallas import tpu_sc as plsc

NC = 2
NS = 16
NW = NC * NS


def _sc_gather_call(idx_flat, table_flat, B, CAT_F, CAT_DIM, CARD1):
    TOT = B * CAT_F
    per_w = TOT // NW
    CHUNKS = 2
    RC = per_w // CHUNKS

    mesh = plsc.VectorSubcoreMesh(core_axis_name="c", subcore_axis_name="s")

    @functools.partial(
        pl.kernel,
        mesh=mesh,
        out_type=jax.ShapeDtypeStruct((TOT, CAT_DIM), jnp.float32),
        compiler_params=pltpu.CompilerParams(use_tc_tiling_on_sc=False),
        scratch_types=[
            pltpu.VMEM((RC,), jnp.int32),
            pltpu.VMEM((RC,), jnp.int32),
            pltpu.VMEM((RC, CAT_DIM), jnp.float32),
            pltpu.SemaphoreType.DMA,
        ],
    )
    def gather_kernel(idx_hbm, table_hbm, out_hbm, idx_v, gidx_v, rows_v, sem):
        wid = lax.axis_index("s") * NC + lax.axis_index("c")
        lane = jnp.arange(16, dtype=jnp.int32)
        for c in range(CHUNKS):
            base = wid * per_w + c * RC
            pltpu.sync_copy(idx_hbm.at[pl.ds(base, RC)], idx_v)

            def body(j, _):
                o = pl.multiple_of(j * 16, 16)
                pos = lane + j * 16
                f = lax.rem(pos, CAT_F)
                gidx_v[pl.ds(o, 16)] = idx_v[pl.ds(o, 16)] + f * CARD1
                return 0

            lax.fori_loop(0, RC // 16, body, 0)
            pltpu.async_copy(table_hbm.at[gidx_v], rows_v, sem).wait()
            pltpu.sync_copy(rows_v, out_hbm.at[pl.ds(base, RC)])

    return gather_kernel(idx_flat, table_flat)


def _tc_numerical_call(vals, W_flat, b_flat, NUM_F, NUM_DIM):
    B = vals.shape[0]
    OUT = NUM_F * NUM_DIM
    BLK = 2048

    def num_kernel(v_ref, w_ref, b_ref, o_ref):
        v = v_ref[...]
        m = jnp.logical_not(jnp.isnan(v))
        mf = m.astype(jnp.float32)
        v0 = jnp.where(m, v, 0.0)
        ci = lax.broadcasted_iota(jnp.int32, (NUM_F, OUT), 1) // NUM_DIM
        ri = lax.broadcasted_iota(jnp.int32, (NUM_F, OUT), 0)
        S = (ci == ri).astype(jnp.float32)
        v_rep = jax.lax.dot(v0, S, precision=lax.Precision.HIGHEST)
        m_rep = jax.lax.dot(mf, S, precision=lax.Precision.HIGHEST)
        o_ref[...] = jnp.maximum(v_rep * w_ref[...] + b_ref[...], 0.0) * m_rep

    return pl.pallas_call(
        num_kernel,
        grid=(B // BLK,),
        in_specs=[
            pl.BlockSpec((BLK, NUM_F), lambda i: (i, 0)),
            pl.BlockSpec((1, OUT), lambda i: (0, 0)),
            pl.BlockSpec((1, OUT), lambda i: (0, 0)),
        ],
        out_specs=pl.BlockSpec((BLK, OUT), lambda i: (i, 0)),
        out_shape=jax.ShapeDtypeStruct((B, OUT), jnp.float32),
    )(vals, W_flat, b_flat)


def _tc_transpose_call(tblT, CAT_F, CARD1, CAT_DIM, CARD_PAD):
    VB = 1024

    def tr_kernel(i_ref, o_ref):
        o_ref[0] = jnp.swapaxes(i_ref[0], 0, 1)

    return pl.pallas_call(
        tr_kernel,
        grid=(CAT_F, CARD_PAD // VB),
        in_specs=[pl.BlockSpec((1, CAT_DIM, VB), lambda f, c: (f, 0, c))],
        out_specs=pl.BlockSpec((1, VB, CAT_DIM), lambda f, c: (f, c, 0)),
        out_shape=jax.ShapeDtypeStruct((CAT_F, CARD_PAD, CAT_DIM), jnp.float32),
    )(tblT)


def kernel(numerical_values, categorical_indices, W_num, b_num, emb_tables):
    B, NUM_F = numerical_values.shape
    CAT_F = categorical_indices.shape[1]
    CARD1 = emb_tables.shape[1]
    CAT_DIM = emb_tables.shape[2]
    NUM_DIM = W_num.shape[1]

    CARD_PAD = (CARD1 + 1023) // 1024 * 1024
    tblT = emb_tables.transpose(0, 2, 1)
    table_rows = _tc_transpose_call(tblT, CAT_F, CARD1, CAT_DIM, CARD_PAD)
    table_flat = table_rows.reshape(CAT_F * CARD_PAD, CAT_DIM)
    idx_flat = categorical_indices.astype(jnp.int32).reshape(B * CAT_F)

    cat = _sc_gather_call(idx_flat, table_flat, B, CAT_F, CAT_DIM, CARD_PAD)
    cat = cat.reshape(B, CAT_F * CAT_DIM)

    W_flat = W_num.reshape(1, NUM_F * NUM_DIM)
    b_flat = b_num.reshape(1, NUM_F * NUM_DIM)
    num = _tc_numerical_call(numerical_values, W_flat, b_flat, NUM_F, NUM_DIM)

    return jnp.concatenate([num, cat], axis=1)


# speedup vs baseline: 5.9345x; 2.2822x over previous
import functools

import jax
import jax.numpy as jnp
from jax import lax
from jax.experimental import pallas as pl
from jax.experimental.pallas import tpu as pltpu
from jax.experimental.pallas import tpu_sc as plsc

NC = 2
NS = 16
NW = NC * NS


def _sc_gather_call(idx_flat, table_flat, B, CAT_F, CAT_DIM, CARD1):
    TOT = B * CAT_F
    per_w = TOT // NW
    CHUNKS = 2
    RC = per_w // CHUNKS

    mesh = plsc.VectorSubcoreMesh(core_axis_name="c", subcore_axis_name="s")

    @functools.partial(
        pl.kernel,
        mesh=mesh,
        out_type=jax.ShapeDtypeStruct((TOT, CAT_DIM), jnp.float32),
        compiler_params=pltpu.CompilerParams(use_tc_tiling_on_sc=False),
        scratch_types=[
            pltpu.VMEM((RC,), jnp.int32),
            pltpu.VMEM((RC,), jnp.int32),
            pltpu.VMEM((RC, CAT_DIM), jnp.float32),
            pltpu.SemaphoreType.DMA,
        ],
    )
    def gather_kernel(idx_hbm, table_hbm, out_hbm, idx_v, gidx_v, rows_v, sem):
        wid = lax.axis_index("s") * NC + lax.axis_index("c")
        lane = jnp.arange(16, dtype=jnp.int32)
        for c in range(CHUNKS):
            base = wid * per_w + c * RC
            pltpu.sync_copy(idx_hbm.at[pl.ds(base, RC)], idx_v)

            def body(j, _):
                o = pl.multiple_of(j * 16, 16)
                pos = lane + j * 16
                f = lax.rem(pos, CAT_F)
                v = idx_v[pl.ds(o, 16)]
                vloc = jnp.bitwise_and(v, 1023)
                p = (
                    jnp.bitwise_and(v, ~1023)
                    + jnp.left_shift(jnp.bitwise_and(vloc, 127), 3)
                    + jnp.right_shift(vloc, 7)
                )
                gidx_v[pl.ds(o, 16)] = p + f * CARD1
                return 0

            lax.fori_loop(0, RC // 16, body, 0)
            pltpu.async_copy(table_hbm.at[gidx_v], rows_v, sem).wait()
            pltpu.sync_copy(rows_v, out_hbm.at[pl.ds(base, RC)])

    return gather_kernel(idx_flat, table_flat)


def _tc_numerical_call(vals, W_flat, b_flat, NUM_F, NUM_DIM):
    B = vals.shape[0]
    OUT = NUM_F * NUM_DIM
    BLK = 2048

    def num_kernel(v_ref, w_ref, b_ref, o_ref):
        v = v_ref[...]
        m = jnp.logical_not(jnp.isnan(v))
        mf = m.astype(jnp.float32)
        v0 = jnp.where(m, v, 0.0)
        ci = lax.broadcasted_iota(jnp.int32, (NUM_F, OUT), 1) // NUM_DIM
        ri = lax.broadcasted_iota(jnp.int32, (NUM_F, OUT), 0)
        S = (ci == ri).astype(jnp.float32)
        v_rep = jax.lax.dot(v0, S, precision=lax.Precision.HIGHEST)
        m_rep = jax.lax.dot(mf, S, precision=lax.Precision.HIGHEST)
        o_ref[...] = jnp.maximum(v_rep * w_ref[...] + b_ref[...], 0.0) * m_rep

    return pl.pallas_call(
        num_kernel,
        grid=(B // BLK,),
        in_specs=[
            pl.BlockSpec((BLK, NUM_F), lambda i: (i, 0)),
            pl.BlockSpec((1, OUT), lambda i: (0, 0)),
            pl.BlockSpec((1, OUT), lambda i: (0, 0)),
        ],
        out_specs=pl.BlockSpec((BLK, OUT), lambda i: (i, 0)),
        out_shape=jax.ShapeDtypeStruct((B, OUT), jnp.float32),
    )(vals, W_flat, b_flat)


def _tc_transpose_call(tblT, CAT_F, CARD1, CAT_DIM, CARD_PAD):
    VB = 2048

    def tr_kernel(i_ref, o_ref):
        for r in range(VB // 1024):
            ts = [
                jnp.swapaxes(
                    i_ref[0, :, (r * 8 + g) * 128:(r * 8 + g + 1) * 128], 0, 1
                )
                for g in range(8)
            ]
            o_ref[0, r * 128:(r + 1) * 128, :] = jnp.concatenate(ts, axis=1)

    return pl.pallas_call(
        tr_kernel,
        grid=(CAT_F, CARD_PAD // VB),
        in_specs=[pl.BlockSpec((1, CAT_DIM, VB), lambda f, c: (f, 0, c))],
        out_specs=pl.BlockSpec((1, VB // 8, 8 * CAT_DIM), lambda f, c: (f, c, 0)),
        out_shape=jax.ShapeDtypeStruct(
            (CAT_F, CARD_PAD // 8, 8 * CAT_DIM), jnp.float32
        ),
    )(tblT)


def kernel(numerical_values, categorical_indices, W_num, b_num, emb_tables):
    B, NUM_F = numerical_values.shape
    CAT_F = categorical_indices.shape[1]
    CARD1 = emb_tables.shape[1]
    CAT_DIM = emb_tables.shape[2]
    NUM_DIM = W_num.shape[1]

    CARD_PAD = (CARD1 + 1023) // 1024 * 1024
    tblT = emb_tables.transpose(0, 2, 1)
    table_rows = _tc_transpose_call(tblT, CAT_F, CARD1, CAT_DIM, CARD_PAD)
    table_flat = table_rows.reshape(CAT_F * CARD_PAD, CAT_DIM)
    idx_flat = categorical_indices.astype(jnp.int32).reshape(B * CAT_F)

    cat = _sc_gather_call(idx_flat, table_flat, B, CAT_F, CAT_DIM, CARD_PAD)
    cat = cat.reshape(B, CAT_F * CAT_DIM)

    W_flat = W_num.reshape(1, NUM_F * NUM_DIM)
    b_flat = b_num.reshape(1, NUM_F * NUM_DIM)
    num = _tc_numerical_call(numerical_values, W_flat, b_flat, NUM_F, NUM_DIM)

    return jnp.concatenate([num, cat], axis=1)
